# baseline (speedup 1.0000x reference)
import functools

import jax
import jax.numpy as jnp
from jax import lax
from jax.experimental import pallas as pl
from jax.experimental.pallas import tpu as pltpu
from jax.experimental.pallas import tpu_sc as plsc

NC = 2
NS = 16
NW = NC * NS
D = 128
DP = D // 2
E_TOTAL = 320000
EPW = E_TOTAL // NW
CHUNK = 80
NROW = 2 * CHUNK
NCHUNK = EPW // CHUNK


def _dot_body(eidx_hbm, z_hbm, out_hbm,
              idx_all, rows_a, rows_b, outv, sem_a, sem_b):
    wid = lax.axis_index("s") * NC + lax.axis_index("c")
    ebase = wid * EPW
    pltpu.sync_copy(eidx_hbm.at[pl.ds(2 * ebase, 2 * EPW)], idx_all)
    lane = lax.iota(jnp.int32, 16)

    def start(c, rows, sem):
        pltpu.async_copy(
            z_hbm.at[idx_all.at[pl.ds(c * NROW, NROW)]], rows, sem)

    def wait(c, rows, sem):
        pltpu.make_async_copy(
            z_hbm.at[idx_all.at[pl.ds(c * NROW, NROW)]], rows, sem).wait()

    def unpack2(v32):
        vbf = plsc.bitcast(v32, jnp.bfloat16)
        return plsc.unpack(vbf, format=plsc.PackFormat.INTERLEAVED)

    def compute(c, rows):
        def group_body(g, carry):
            ei2 = (lane + g * 16) * 2
            ej2 = ei2 + 1
            accs = [jnp.zeros((16,), jnp.float32) for _ in range(2)]
            for blk in range(DP // 16):
                for t in range(16):
                    dv = (lane ^ t) + blk * 16
                    via, vib = unpack2(plsc.load_gather(rows, [ei2, dv]))
                    vja, vjb = unpack2(plsc.load_gather(rows, [ej2, dv]))
                    accs[0] = accs[0] + via * vja
                    accs[1] = accs[1] + vib * vjb
            outv[pl.ds(c * CHUNK + g * 16, 16)] = accs[0] + accs[1]
            return carry

        lax.fori_loop(0, CHUNK // 16, group_body, 0)

    start(0, rows_a, sem_a)

    def body2(t2, carry):
        t = 2 * t2
        start(t + 1, rows_b, sem_b)
        wait(t, rows_a, sem_a)
        compute(t, rows_a)
        start(t + 2, rows_a, sem_a)
        wait(t + 1, rows_b, sem_b)
        compute(t + 1, rows_b)
        return carry

    lax.fori_loop(0, (NCHUNK - 1) // 2, body2, 0)
    wait(NCHUNK - 1, rows_a, sem_a)
    compute(NCHUNK - 1, rows_a)
    pltpu.sync_copy(outv, out_hbm.at[pl.ds(ebase, EPW)])


@jax.jit
def kernel(z, edge_index):
    eidx2 = jnp.stack(
        [edge_index[0].astype(jnp.int32), edge_index[1].astype(jnp.int32)],
        axis=1).reshape(-1)
    zp = lax.bitcast_convert_type(
        z.astype(jnp.bfloat16).reshape(z.shape[0], DP, 2), jnp.int32)
    mesh = plsc.VectorSubcoreMesh(core_axis_name="c", subcore_axis_name="s")
    f = functools.partial(
        pl.kernel,
        mesh=mesh,
        out_type=jax.ShapeDtypeStruct((E_TOTAL,), jnp.float32),
        scratch_types=[
            pltpu.VMEM((2 * EPW,), jnp.int32),
            pltpu.VMEM((NROW, DP), jnp.int32),
            pltpu.VMEM((NROW, DP), jnp.int32),
            pltpu.VMEM((EPW,), jnp.float32),
            pltpu.SemaphoreType.DMA,
            pltpu.SemaphoreType.DMA,
        ],
        compiler_params=pltpu.CompilerParams(
            needs_layout_passes=False, use_tc_tiling_on_sc=False),
    )(_dot_body)
    return f(eidx2, zp)


# speedup vs baseline: 3.2971x; 3.2971x over previous
import functools

import jax
import jax.numpy as jnp
from jax import lax
from jax.experimental import pallas as pl
from jax.experimental.pallas import tpu as pltpu
from jax.experimental.pallas import tpu_sc as plsc

NC = 2
NS = 16
NW = NC * NS
D = 128
DP = D // 2
E_TOTAL = 320000
EPW = E_TOTAL // NW
CHUNK = 80
NCHUNK = EPW // CHUNK


def _dot_body(ei_hbm, ej_hbm, z_hbm, out_hbm,
              idxi_all, idxj_all, ri_a, rj_a, ri_b, rj_b, outv,
              si_a, sj_a, si_b, sj_b):
    wid = lax.axis_index("s") * NC + lax.axis_index("c")
    ebase = wid * EPW
    pltpu.sync_copy(ei_hbm.at[pl.ds(ebase, EPW)], idxi_all)
    pltpu.sync_copy(ej_hbm.at[pl.ds(ebase, EPW)], idxj_all)
    lane = lax.iota(jnp.int32, 16)

    def start(c, ri, rj, si, sj):
        pltpu.async_copy(z_hbm.at[idxi_all.at[pl.ds(c * CHUNK, CHUNK)]], ri, si)
        pltpu.async_copy(z_hbm.at[idxj_all.at[pl.ds(c * CHUNK, CHUNK)]], rj, sj)

    def wait(c, ri, rj, si, sj):
        pltpu.make_async_copy(
            z_hbm.at[idxi_all.at[pl.ds(c * CHUNK, CHUNK)]], ri, si).wait()
        pltpu.make_async_copy(
            z_hbm.at[idxj_all.at[pl.ds(c * CHUNK, CHUNK)]], rj, sj).wait()

    def pair_prod(vi32, vj32):
        pbf = plsc.bitcast(vi32, jnp.bfloat16) * plsc.bitcast(vj32, jnp.bfloat16)
        return plsc.unpack(pbf, format=plsc.PackFormat.INTERLEAVED)

    def compute(c, ri, rj):
        def group_body(g, carry):
            e_idx = lane + g * 16
            accs = [jnp.zeros((16,), jnp.float32) for _ in range(2)]
            for blk in range(DP // 16):
                for t in range(16):
                    dv = (lane ^ t) + blk * 16
                    pa, pb = pair_prod(plsc.load_gather(ri, [e_idx, dv]),
                                       plsc.load_gather(rj, [e_idx, dv]))
                    accs[0] = accs[0] + pa
                    accs[1] = accs[1] + pb
            outv[pl.ds(c * CHUNK + g * 16, 16)] = accs[0] + accs[1]
            return carry

        lax.fori_loop(0, CHUNK // 16, group_body, 0)

    start(0, ri_a, rj_a, si_a, sj_a)

    def body2(t2, carry):
        t = 2 * t2
        start(t + 1, ri_b, rj_b, si_b, sj_b)
        wait(t, ri_a, rj_a, si_a, sj_a)
        compute(t, ri_a, rj_a)
        start(t + 2, ri_a, rj_a, si_a, sj_a)
        wait(t + 1, ri_b, rj_b, si_b, sj_b)
        compute(t + 1, ri_b, rj_b)
        return carry

    lax.fori_loop(0, (NCHUNK - 1) // 2, body2, 0)
    wait(NCHUNK - 1, ri_a, rj_a, si_a, sj_a)
    compute(NCHUNK - 1, ri_a, rj_a)
    pltpu.sync_copy(outv, out_hbm.at[pl.ds(ebase, EPW)])


@jax.jit
def kernel(z, edge_index):
    ei = edge_index[0].astype(jnp.int32)
    ej = edge_index[1].astype(jnp.int32)
    zp = lax.bitcast_convert_type(
        z.astype(jnp.bfloat16).reshape(z.shape[0], DP, 2), jnp.int32)
    mesh = plsc.VectorSubcoreMesh(core_axis_name="c", subcore_axis_name="s")
    f = functools.partial(
        pl.kernel,
        mesh=mesh,
        out_type=jax.ShapeDtypeStruct((E_TOTAL,), jnp.float32),
        scratch_types=[
            pltpu.VMEM((EPW,), jnp.int32),
            pltpu.VMEM((EPW,), jnp.int32),
            pltpu.VMEM((CHUNK, DP), jnp.int32),
            pltpu.VMEM((CHUNK, DP), jnp.int32),
            pltpu.VMEM((CHUNK, DP), jnp.int32),
            pltpu.VMEM((CHUNK, DP), jnp.int32),
            pltpu.VMEM((EPW,), jnp.float32),
            pltpu.SemaphoreType.DMA,
            pltpu.SemaphoreType.DMA,
            pltpu.SemaphoreType.DMA,
            pltpu.SemaphoreType.DMA,
        ],
        compiler_params=pltpu.CompilerParams(
            needs_layout_passes=False, use_tc_tiling_on_sc=False),
    )(_dot_body)
    return f(ei, ej, zp)
